# Initial kernel scaffold; baseline (speedup 1.0000x reference)
import jax
import jax.numpy as jnp
from jax.experimental import pallas as pl


def kernel(x, edge_index, W1, b1, W2, b2, W3, b3):
    raise NotImplementedError("write your pallas kernel here")


# speedup vs baseline: 9.1108x; 9.1108x over previous
import functools

import jax
import jax.numpy as jnp
from jax import lax
from jax.experimental import pallas as pl
from jax.experimental.pallas import tpu as pltpu
from jax.experimental.pallas import tpu_sc as plsc

N = 10000
E = 320000
D = 128
NCLS = 40

NC = 2
NS = 16
EPC = E // NC
EPW = EPC // NS
K = 80
STEPS = EPW // K
ZCH = 624
TAIL = N - NS * ZCH
ZB = 208
DEGW = 16

_MESH = plsc.VectorSubcoreMesh(
    core_axis_name="c", subcore_axis_name="s", num_cores=NC, num_subcores=NS
)


def _fill_rows(ref, rows, width, value_row):
    per_row = width // 16

    def body(t, carry):
        i = t // per_row
        j = t % per_row
        ref[i, pl.ds(j * 16, 16)] = value_row
        return carry

    lax.fori_loop(0, rows * per_row, body, 0)


@functools.partial(
    pl.kernel,
    out_type=jax.ShapeDtypeStruct((NC, N, D), jnp.float32),
    mesh=_MESH,
    scratch_types=[
        pltpu.VMEM((K,), jnp.int32),
        pltpu.VMEM((K,), jnp.int32),
        pltpu.VMEM((K, D), jnp.float32),
        pltpu.VMEM((ZB, D), jnp.float32),
        pltpu.VMEM_SHARED((N, D), jnp.float32),
        pltpu.SemaphoreType.DMA,
    ],
)
def _sc_aggregate(u_hbm, src_hbm, dst_hbm, out_hbm, idx_s, idx_d, rows_v, zbuf_v, acc_sh, sem):
    c = lax.axis_index("c")
    s = lax.axis_index("s")
    zero_row = jnp.zeros((16,), jnp.float32)
    _fill_rows(zbuf_v, ZB, D, zero_row)
    for z in range(ZCH // ZB):
        pltpu.sync_copy(zbuf_v, acc_sh.at[pl.ds(s * ZCH + z * ZB, ZB)])

    @pl.when(s == 0)
    def _():
        pltpu.sync_copy(zbuf_v.at[pl.ds(0, TAIL)], acc_sh.at[pl.ds(NS * ZCH, TAIL)])

    plsc.subcore_barrier()
    base = c * EPC + s * EPW

    def step(t, carry):
        pltpu.sync_copy(src_hbm.at[pl.ds(base + t * K, K)], idx_s)
        pltpu.sync_copy(dst_hbm.at[pl.ds(base + t * K, K)], idx_d)
        pltpu.async_copy(u_hbm.at[idx_s], rows_v, sem).wait()
        pltpu.sync_copy(rows_v, acc_sh.at[idx_d], add=True)
        return carry

    lax.fori_loop(0, STEPS, step, 0)
    plsc.subcore_barrier()
    pltpu.sync_copy(acc_sh.at[pl.ds(s * ZCH, ZCH)], out_hbm.at[c, pl.ds(s * ZCH, ZCH)])

    @pl.when(s == 0)
    def _():
        pltpu.sync_copy(acc_sh.at[pl.ds(NS * ZCH, TAIL)], out_hbm.at[c, pl.ds(NS * ZCH, TAIL)])


_BN = 2000


def _prep_body(degp_ref, x_ref, dinv_ref, u_ref):
    d = degp_ref[0, :, 0:1] + degp_ref[1, :, 0:1] + 1.0
    dv = lax.rsqrt(d)
    dinv_ref[...] = dv
    u_ref[...] = dv * x_ref[...]


def _tc_prep(degp, x):
    return pl.pallas_call(
        _prep_body,
        grid=(N // _BN,),
        in_specs=[
            pl.BlockSpec((NC, _BN, D), lambda i: (0, i, 0)),
            pl.BlockSpec((_BN, D), lambda i: (i, 0)),
        ],
        out_specs=[
            pl.BlockSpec((_BN, 1), lambda i: (i, 0)),
            pl.BlockSpec((_BN, D), lambda i: (i, 0)),
        ],
        out_shape=[
            jax.ShapeDtypeStruct((N, 1), jnp.float32),
            jax.ShapeDtypeStruct((N, D), jnp.float32),
        ],
    )(degp, x)


def _layer_body(p_ref, u_ref, dinv_ref, w_ref, b_ref, o_ref, *, scale_out):
    dv = dinv_ref[...]
    agg = (p_ref[0] + p_ref[1] + u_ref[...]) * dv
    h = jnp.dot(agg, w_ref[...], preferred_element_type=jnp.float32) + b_ref[...]
    if scale_out:
        h = jnp.maximum(h, 0.0) * dv
    o_ref[...] = h


def _tc_layer(p, u, dinv, w, b, scale_out):
    return pl.pallas_call(
        functools.partial(_layer_body, scale_out=scale_out),
        grid=(N // _BN,),
        in_specs=[
            pl.BlockSpec((NC, _BN, D), lambda i: (0, i, 0)),
            pl.BlockSpec((_BN, D), lambda i: (i, 0)),
            pl.BlockSpec((_BN, 1), lambda i: (i, 0)),
            pl.BlockSpec((D, D), lambda i: (0, 0)),
            pl.BlockSpec((1, D), lambda i: (0, 0)),
        ],
        out_specs=pl.BlockSpec((_BN, D), lambda i: (i, 0)),
        out_shape=jax.ShapeDtypeStruct((N, D), jnp.float32),
    )(p, u, dinv, w, b)


def kernel(x, edge_index, W1, b1, W2, b2, W3, b3):
    src = edge_index[0]
    dst = edge_index[1]
    e1_table = jnp.zeros((N, D), jnp.float32).at[:, 0].set(1.0)
    degp = _sc_aggregate(e1_table, dst, dst)
    dinv, u1 = _tc_prep(degp, x)
    p1 = _sc_aggregate(u1, src, dst)
    u2 = _tc_layer(p1, u1, dinv, W1, jnp.reshape(b1, (1, D)), True)
    p2 = _sc_aggregate(u2, src, dst)
    u3 = _tc_layer(p2, u2, dinv, W2, jnp.reshape(b2, (1, D)), True)
    p3 = _sc_aggregate(u3, src, dst)
    w3p = jnp.zeros((D, D), jnp.float32).at[:, :NCLS].set(W3)
    b3p = jnp.zeros((1, D), jnp.float32).at[0, :NCLS].set(b3)
    out = _tc_layer(p3, u3, dinv, w3p, b3p, False)
    return out[:, :NCLS]


# speedup vs baseline: 22.6824x; 2.4896x over previous
import functools

import jax
import jax.numpy as jnp
from jax import lax
from jax.experimental import pallas as pl
from jax.experimental.pallas import tpu as pltpu
from jax.experimental.pallas import tpu_sc as plsc

N = 10000
E = 320000
D = 128
NCLS = 40

NC = 2
NS = 16
NW = NC * NS
EPW = E // NW
K = 50
STEPS = EPW // K
NPAIRS = STEPS // 2
NBUF = 4
PF = 2
OCT = 8
ZCH = 624
TAIL = N - NS * ZCH
ZB = 48
FIRE = 25

_MESH = plsc.VectorSubcoreMesh(
    core_axis_name="c", subcore_axis_name="s", num_cores=NC, num_subcores=NS
)


def _fill_rows(ref, rows, width, value_row):
    per_row = width // 16

    def body(t, carry):
        i = t // per_row
        j = t % per_row
        ref[i, pl.ds(j * 16, 16)] = value_row
        return carry

    lax.fori_loop(0, rows * per_row, body, 0)


def _zero_acc(zsrc, acc_sh, s):
    for z in range(ZCH // ZB):
        pltpu.sync_copy(zsrc.at[pl.ds(0, ZB)], acc_sh.at[pl.ds(s * ZCH + z * ZB, ZB)])

    @pl.when(s == 0)
    def _():
        pltpu.sync_copy(zsrc.at[pl.ds(0, TAIL)], acc_sh.at[pl.ds(NS * ZCH, TAIL)])


def _copy_out(acc_sh, out_hbm, c, s):
    pltpu.sync_copy(acc_sh.at[pl.ds(s * ZCH, ZCH)], out_hbm.at[c, pl.ds(s * ZCH, ZCH)])

    @pl.when(s == 0)
    def _():
        pltpu.sync_copy(acc_sh.at[pl.ds(NS * ZCH, TAIL)], out_hbm.at[c, pl.ds(NS * ZCH, TAIL)])


@functools.partial(
    pl.kernel,
    out_type=jax.ShapeDtypeStruct((NC, N, D), jnp.float32),
    mesh=_MESH,
    scratch_types=[
        pltpu.VMEM((4, 2, 2, K), jnp.int32),
        pltpu.VMEM((NBUF, K, D), jnp.float32),
        pltpu.VMEM_SHARED((N, D), jnp.float32),
        pltpu.SemaphoreType.DMA,
        pltpu.SemaphoreType.DMA,
        pltpu.SemaphoreType.DMA,
        pltpu.SemaphoreType.DMA,
        pltpu.SemaphoreType.DMA,
        pltpu.SemaphoreType.DMA,
        pltpu.SemaphoreType.DMA,
        pltpu.SemaphoreType.DMA,
        pltpu.SemaphoreType.DMA,
        pltpu.SemaphoreType.DMA,
        pltpu.SemaphoreType.DMA,
        pltpu.SemaphoreType.DMA,
    ],
)
def _sc_aggregate(
    u_hbm, idx3_hbm, out_hbm,
    iring, rows_v, acc_sh,
    i0, i1, i2, i3, g0, g1, g2, g3, s0, s1, s2, s3,
):
    c = lax.axis_index("c")
    s = lax.axis_index("s")
    w = c * NS + s
    isem = (i0, i1, i2, i3)
    gsem = (g0, g1, g2, g3)
    ssem = (s0, s1, s2, s3)

    zero_row = jnp.zeros((16,), jnp.float32)

    def zfill(t, carry):
        rows_v[0, t // 8, pl.ds((t % 8) * 16, 16)] = zero_row
        return carry

    lax.fori_loop(0, ZB * 8, zfill, 0)
    _zero_acc(rows_v.at[0], acc_sh, s)
    plsc.subcore_barrier()

    def icopy(j, jslot):
        return pltpu.make_async_copy(
            idx3_hbm.at[w, pl.ds(j * 2, 2)], iring.at[jslot], isem[jslot]
        )

    def gather(rb, ps, par):
        return pltpu.make_async_copy(
            u_hbm.at[iring.at[ps, par, 0]], rows_v.at[rb], gsem[rb]
        )

    def scat(rb, ps, par):
        return pltpu.make_async_copy(
            rows_v.at[rb], acc_sh.at[iring.at[ps, par, 1]], ssem[rb]
        )

    icopy(0, 0).start()
    icopy(0, 0).wait()
    icopy(1, 1).start()
    gather(0, 0, 0).start()
    gather(1, 0, 1).start()

    def octet(i, carry):
        t0 = i * OCT
        for b8 in range(OCT):
            t = t0 + b8
            rb = b8 % NBUF
            ps = (b8 // 2) % 4
            par = b8 % 2
            if par == 0:
                j = t // 2

                @pl.when(j + 1 < NPAIRS)
                def _():
                    icopy(j + 1, (b8 // 2 + 1) % 4).wait()

                @pl.when(j + 2 < NPAIRS)
                def _():
                    icopy(j + 2, (b8 // 2 + 2) % 4).start()

            gather(rb, ps, par).wait()
            scat(rb, ps, par).start(add=True)

            @pl.when(t >= PF)
            def _():
                scat((b8 - PF) % NBUF, ((b8 - PF) // 2) % 4, par).wait()

            @pl.when(t + PF < STEPS)
            def _():
                gather((b8 + PF) % NBUF, ((b8 + PF) // 2) % 4, par).start()

        return carry

    lax.fori_loop(0, STEPS // OCT, octet, 0)
    scat((STEPS - PF) % NBUF, ((STEPS - PF) // 2) % 4, (STEPS - PF) % 2).wait()
    scat((STEPS - 1) % NBUF, ((STEPS - 1) // 2) % 4, (STEPS - 1) % 2).wait()
    plsc.subcore_barrier()
    _copy_out(acc_sh, out_hbm, c, s)


@functools.partial(
    pl.kernel,
    out_type=jax.ShapeDtypeStruct((NC, N, D), jnp.float32),
    mesh=_MESH,
    scratch_types=[
        pltpu.VMEM((STEPS, K), jnp.int32),
        pltpu.VMEM((K, D), jnp.float32),
        pltpu.VMEM_SHARED((N, D), jnp.float32),
        pltpu.SemaphoreType.DMA,
    ],
)
def _sc_degree(dst3_hbm, out_hbm, dst_all, ones_v, acc_sh, s0):
    c = lax.axis_index("c")
    s = lax.axis_index("s")
    w = c * NS + s
    zero_row = jnp.zeros((16,), jnp.float32)
    e1_row = jnp.where(lax.iota(jnp.int32, 16) == 0, 1.0, 0.0).astype(jnp.float32)

    _fill_rows(ones_v, K, D, zero_row)
    _zero_acc(ones_v, acc_sh, s)

    def set_e1(i, carry):
        ones_v[i, pl.ds(0, 16)] = e1_row
        return carry

    lax.fori_loop(0, K, set_e1, 0)
    pltpu.sync_copy(dst3_hbm.at[w], dst_all)
    plsc.subcore_barrier()

    def scat(t):
        return pltpu.make_async_copy(ones_v, acc_sh.at[dst_all.at[t]], s0)

    def batch(i, carry):
        t0 = i * FIRE

        def fire(t, cc):
            scat(t).start(add=True)
            return cc

        def drain(t, cc):
            scat(t).wait()
            return cc

        lax.fori_loop(t0, t0 + FIRE, fire, 0)
        lax.fori_loop(t0, t0 + FIRE, drain, 0)
        return carry

    lax.fori_loop(0, STEPS // FIRE, batch, 0)
    plsc.subcore_barrier()
    _copy_out(acc_sh, out_hbm, c, s)


_BN = 2000


def _prep_body(degp_ref, x_ref, dinv_ref, u_ref):
    d = degp_ref[0, :, 0:1] + degp_ref[1, :, 0:1] + 1.0
    dv = lax.rsqrt(d)
    dinv_ref[...] = dv
    u_ref[...] = dv * x_ref[...]


def _tc_prep(degp, x):
    return pl.pallas_call(
        _prep_body,
        grid=(N // _BN,),
        in_specs=[
            pl.BlockSpec((NC, _BN, D), lambda i: (0, i, 0)),
            pl.BlockSpec((_BN, D), lambda i: (i, 0)),
        ],
        out_specs=[
            pl.BlockSpec((_BN, 1), lambda i: (i, 0)),
            pl.BlockSpec((_BN, D), lambda i: (i, 0)),
        ],
        out_shape=[
            jax.ShapeDtypeStruct((N, 1), jnp.float32),
            jax.ShapeDtypeStruct((N, D), jnp.float32),
        ],
    )(degp, x)


def _layer_body(p_ref, u_ref, dinv_ref, w_ref, b_ref, o_ref, *, scale_out):
    dv = dinv_ref[...]
    agg = (p_ref[0] + p_ref[1] + u_ref[...]) * dv
    h = jnp.dot(agg, w_ref[...], preferred_element_type=jnp.float32) + b_ref[...]
    if scale_out:
        h = jnp.maximum(h, 0.0) * dv
    o_ref[...] = h


def _tc_layer(p, u, dinv, w, b, scale_out):
    return pl.pallas_call(
        functools.partial(_layer_body, scale_out=scale_out),
        grid=(N // _BN,),
        in_specs=[
            pl.BlockSpec((NC, _BN, D), lambda i: (0, i, 0)),
            pl.BlockSpec((_BN, D), lambda i: (i, 0)),
            pl.BlockSpec((_BN, 1), lambda i: (i, 0)),
            pl.BlockSpec((D, D), lambda i: (0, 0)),
            pl.BlockSpec((1, D), lambda i: (0, 0)),
        ],
        out_specs=pl.BlockSpec((_BN, D), lambda i: (i, 0)),
        out_shape=jax.ShapeDtypeStruct((N, D), jnp.float32),
    )(p, u, dinv, w, b)


def kernel(x, edge_index, W1, b1, W2, b2, W3, b3):
    src3 = jnp.reshape(edge_index[0], (NW, STEPS, K))
    dst3 = jnp.reshape(edge_index[1], (NW, STEPS, K))
    idx3 = jnp.stack([src3, dst3], axis=2)
    degp = _sc_degree(dst3)
    dinv, u1 = _tc_prep(degp, x)
    p1 = _sc_aggregate(u1, idx3)
    u2 = _tc_layer(p1, u1, dinv, W1, jnp.reshape(b1, (1, D)), True)
    p2 = _sc_aggregate(u2, idx3)
    u3 = _tc_layer(p2, u2, dinv, W2, jnp.reshape(b2, (1, D)), True)
    p3 = _sc_aggregate(u3, idx3)
    w3p = jnp.zeros((D, D), jnp.float32).at[:, :NCLS].set(W3)
    b3p = jnp.zeros((1, D), jnp.float32).at[0, :NCLS].set(b3)
    out = _tc_layer(p3, u3, dinv, w3p, b3p, False)
    return out[:, :NCLS]


# speedup vs baseline: 27.1687x; 1.1978x over previous
import functools

import jax
import jax.numpy as jnp
from jax import lax
from jax.experimental import pallas as pl
from jax.experimental.pallas import tpu as pltpu
from jax.experimental.pallas import tpu_sc as plsc

N = 10000
E = 320000
D = 128
NCLS = 40

NC = 2
NS = 16
NW = NC * NS
EPW = E // NW
K = 50
STEPS = EPW // K
NPAIRS = STEPS // 2
NBUF = 4
PF = 3
OCT = 8
ZCH = 624
TAIL = N - NS * ZCH
ZB = 48
FIRE = 25

_MESH = plsc.VectorSubcoreMesh(
    core_axis_name="c", subcore_axis_name="s", num_cores=NC, num_subcores=NS
)


def _fill_rows(ref, rows, width, value_row):
    per_row = width // 16

    def body(t, carry):
        i = t // per_row
        j = t % per_row
        ref[i, pl.ds(j * 16, 16)] = value_row
        return carry

    lax.fori_loop(0, rows * per_row, body, 0)


def _zero_acc(zsrc, acc_sh, s):
    for z in range(ZCH // ZB):
        pltpu.sync_copy(zsrc.at[pl.ds(0, ZB)], acc_sh.at[pl.ds(s * ZCH + z * ZB, ZB)])

    @pl.when(s == 0)
    def _():
        pltpu.sync_copy(zsrc.at[pl.ds(0, TAIL)], acc_sh.at[pl.ds(NS * ZCH, TAIL)])


def _copy_out(acc_sh, out_hbm, c, s):
    pltpu.sync_copy(acc_sh.at[pl.ds(s * ZCH, ZCH)], out_hbm.at[c, pl.ds(s * ZCH, ZCH)])

    @pl.when(s == 0)
    def _():
        pltpu.sync_copy(acc_sh.at[pl.ds(NS * ZCH, TAIL)], out_hbm.at[c, pl.ds(NS * ZCH, TAIL)])


@functools.partial(
    pl.kernel,
    out_type=jax.ShapeDtypeStruct((NC, N, D), jnp.float32),
    mesh=_MESH,
    scratch_types=[
        pltpu.VMEM((4, 2, 2, K), jnp.int32),
        pltpu.VMEM((NBUF, K, D), jnp.float32),
        pltpu.VMEM_SHARED((N, D), jnp.float32),
        pltpu.SemaphoreType.DMA,
        pltpu.SemaphoreType.DMA,
        pltpu.SemaphoreType.DMA,
        pltpu.SemaphoreType.DMA,
        pltpu.SemaphoreType.DMA,
        pltpu.SemaphoreType.DMA,
        pltpu.SemaphoreType.DMA,
        pltpu.SemaphoreType.DMA,
        pltpu.SemaphoreType.DMA,
        pltpu.SemaphoreType.DMA,
        pltpu.SemaphoreType.DMA,
        pltpu.SemaphoreType.DMA,
    ],
)
def _sc_aggregate(
    u_hbm, idx3_hbm, out_hbm,
    iring, rows_v, acc_sh,
    i0, i1, i2, i3, g0, g1, g2, g3, s0, s1, s2, s3,
):
    c = lax.axis_index("c")
    s = lax.axis_index("s")
    w = c * NS + s
    isem = (i0, i1, i2, i3)
    gsem = (g0, g1, g2, g3)
    ssem = (s0, s1, s2, s3)

    zero_row = jnp.zeros((16,), jnp.float32)

    def icopy(j, jslot):
        return pltpu.make_async_copy(
            idx3_hbm.at[w, pl.ds(j * 2, 2)], iring.at[jslot], isem[jslot]
        )

    def gather(rb, ps, par):
        return pltpu.make_async_copy(
            u_hbm.at[iring.at[ps, par, 0]], rows_v.at[rb], gsem[rb]
        )

    def scat(rb, ps, par):
        return pltpu.make_async_copy(
            rows_v.at[rb], acc_sh.at[iring.at[ps, par, 1]], ssem[rb]
        )

    icopy(0, 0).start()
    icopy(1, 1).start()
    icopy(2, 2).start()
    icopy(0, 0).wait()
    gather(0, 0, 0).start()
    gather(1, 0, 1).start()
    icopy(1, 1).wait()
    gather(2, 1, 0).start()

    def zfill(t, carry):
        rows_v[3, t // 8, pl.ds((t % 8) * 16, 16)] = zero_row
        return carry

    lax.fori_loop(0, ZB * 8, zfill, 0)
    _zero_acc(rows_v.at[3], acc_sh, s)
    plsc.subcore_barrier()

    def octet(i, carry):
        t0 = i * OCT
        for b8 in range(OCT):
            t = t0 + b8
            rb = b8 % NBUF
            ps = (b8 // 2) % 4
            par = b8 % 2
            gather(rb, ps, par).wait()
            scat(rb, ps, par).start(add=True)

            @pl.when(t >= 1)
            def _():
                scat((b8 - 1) % NBUF, ((b8 - 1) // 2) % 4, (b8 - 1) % 2).wait()

            if par == 0:
                j = t // 2

                @pl.when(j + 2 < NPAIRS)
                def _():
                    icopy(j + 2, (b8 // 2 + 2) % 4).wait()

                @pl.when(j + 3 < NPAIRS)
                def _():
                    icopy(j + 3, (b8 // 2 + 3) % 4).start()

            @pl.when(t + PF < STEPS)
            def _():
                gather((b8 + PF) % NBUF, ((b8 + PF) // 2) % 4, (b8 + PF) % 2).start()

        return carry

    lax.fori_loop(0, STEPS // OCT, octet, 0)
    scat((STEPS - 1) % NBUF, ((STEPS - 1) // 2) % 4, (STEPS - 1) % 2).wait()
    plsc.subcore_barrier()
    _copy_out(acc_sh, out_hbm, c, s)


@functools.partial(
    pl.kernel,
    out_type=jax.ShapeDtypeStruct((NC, N, D), jnp.float32),
    mesh=_MESH,
    scratch_types=[
        pltpu.VMEM((STEPS, K), jnp.int32),
        pltpu.VMEM((K, D), jnp.float32),
        pltpu.VMEM_SHARED((N, D), jnp.float32),
        pltpu.SemaphoreType.DMA,
    ],
)
def _sc_degree(dst3_hbm, out_hbm, dst_all, ones_v, acc_sh, s0):
    c = lax.axis_index("c")
    s = lax.axis_index("s")
    w = c * NS + s
    zero_row = jnp.zeros((16,), jnp.float32)
    e1_row = jnp.where(lax.iota(jnp.int32, 16) == 0, 1.0, 0.0).astype(jnp.float32)

    _fill_rows(ones_v, K, D, zero_row)
    _zero_acc(ones_v, acc_sh, s)

    def set_e1(i, carry):
        ones_v[i, pl.ds(0, 16)] = e1_row
        return carry

    lax.fori_loop(0, K, set_e1, 0)
    pltpu.sync_copy(dst3_hbm.at[w], dst_all)
    plsc.subcore_barrier()

    def scat(t):
        return pltpu.make_async_copy(ones_v, acc_sh.at[dst_all.at[t]], s0)

    def batch(i, carry):
        t0 = i * FIRE

        def fire(t, cc):
            scat(t).start(add=True)
            return cc

        def drain(t, cc):
            scat(t).wait()
            return cc

        lax.fori_loop(t0, t0 + FIRE, fire, 0)
        lax.fori_loop(t0, t0 + FIRE, drain, 0)
        return carry

    lax.fori_loop(0, STEPS // FIRE, batch, 0)
    plsc.subcore_barrier()
    _copy_out(acc_sh, out_hbm, c, s)


_BN = 2000


def _prep_body(degp_ref, x_ref, dinv_ref, u_ref):
    d = degp_ref[0, :, 0:1] + degp_ref[1, :, 0:1] + 1.0
    dv = lax.rsqrt(d)
    dinv_ref[...] = dv
    u_ref[...] = dv * x_ref[...]


def _tc_prep(degp, x):
    return pl.pallas_call(
        _prep_body,
        grid=(N // _BN,),
        in_specs=[
            pl.BlockSpec((NC, _BN, D), lambda i: (0, i, 0)),
            pl.BlockSpec((_BN, D), lambda i: (i, 0)),
        ],
        out_specs=[
            pl.BlockSpec((_BN, 1), lambda i: (i, 0)),
            pl.BlockSpec((_BN, D), lambda i: (i, 0)),
        ],
        out_shape=[
            jax.ShapeDtypeStruct((N, 1), jnp.float32),
            jax.ShapeDtypeStruct((N, D), jnp.float32),
        ],
    )(degp, x)


def _layer_body(p_ref, u_ref, dinv_ref, w_ref, b_ref, o_ref, *, scale_out):
    dv = dinv_ref[...]
    agg = (p_ref[0] + p_ref[1] + u_ref[...]) * dv
    h = jnp.dot(agg, w_ref[...], preferred_element_type=jnp.float32) + b_ref[...]
    if scale_out:
        h = jnp.maximum(h, 0.0) * dv
    o_ref[...] = h


def _tc_layer(p, u, dinv, w, b, scale_out):
    return pl.pallas_call(
        functools.partial(_layer_body, scale_out=scale_out),
        grid=(N // _BN,),
        in_specs=[
            pl.BlockSpec((NC, _BN, D), lambda i: (0, i, 0)),
            pl.BlockSpec((_BN, D), lambda i: (i, 0)),
            pl.BlockSpec((_BN, 1), lambda i: (i, 0)),
            pl.BlockSpec((D, D), lambda i: (0, 0)),
            pl.BlockSpec((1, D), lambda i: (0, 0)),
        ],
        out_specs=pl.BlockSpec((_BN, D), lambda i: (i, 0)),
        out_shape=jax.ShapeDtypeStruct((N, D), jnp.float32),
    )(p, u, dinv, w, b)


def kernel(x, edge_index, W1, b1, W2, b2, W3, b3):
    src3 = jnp.reshape(edge_index[0], (NW, STEPS, K))
    dst3 = jnp.reshape(edge_index[1], (NW, STEPS, K))
    idx3 = jnp.stack([src3, dst3], axis=2)
    degp = _sc_degree(dst3)
    dinv, u1 = _tc_prep(degp, x)
    p1 = _sc_aggregate(u1, idx3)
    u2 = _tc_layer(p1, u1, dinv, W1, jnp.reshape(b1, (1, D)), True)
    p2 = _sc_aggregate(u2, idx3)
    u3 = _tc_layer(p2, u2, dinv, W2, jnp.reshape(b2, (1, D)), True)
    p3 = _sc_aggregate(u3, idx3)
    w3p = jnp.zeros((D, D), jnp.float32).at[:, :NCLS].set(W3)
    b3p = jnp.zeros((1, D), jnp.float32).at[0, :NCLS].set(b3)
    out = _tc_layer(p3, u3, dinv, w3p, b3p, False)
    return out[:, :NCLS]


# speedup vs baseline: 27.2689x; 1.0037x over previous
import functools

import jax
import jax.numpy as jnp
from jax import lax
from jax.experimental import pallas as pl
from jax.experimental.pallas import tpu as pltpu
from jax.experimental.pallas import tpu_sc as plsc

N = 10000
E = 320000
D = 128
NCLS = 40

NC = 2
NS = 16
NW = NC * NS
EPW = E // NW
K = 50
STEPS = EPW // K
NPAIRS = STEPS // 2
NBUF = 4
PF = 3
OCT = 8
ZCH = 624
TAIL = N - NS * ZCH
ZB = 48
KD = 100
STEPSD = EPW // KD
FIRE = 25

_MESH = plsc.VectorSubcoreMesh(
    core_axis_name="c", subcore_axis_name="s", num_cores=NC, num_subcores=NS
)


def _fill_rows(ref, rows, width, value_row):
    per_row = width // 16

    def body(t, carry):
        i = t // per_row
        j = t % per_row
        ref[i, pl.ds(j * 16, 16)] = value_row
        return carry

    lax.fori_loop(0, rows * per_row, body, 0)


def _zero_acc(zsrc, acc_sh, s):
    for z in range(ZCH // ZB):
        pltpu.sync_copy(zsrc.at[pl.ds(0, ZB)], acc_sh.at[pl.ds(s * ZCH + z * ZB, ZB)])

    @pl.when(s == 0)
    def _():
        pltpu.sync_copy(zsrc.at[pl.ds(0, TAIL)], acc_sh.at[pl.ds(NS * ZCH, TAIL)])


def _copy_out(acc_sh, out_hbm, c, s):
    pltpu.sync_copy(acc_sh.at[pl.ds(s * ZCH, ZCH)], out_hbm.at[c, pl.ds(s * ZCH, ZCH)])

    @pl.when(s == 0)
    def _():
        pltpu.sync_copy(acc_sh.at[pl.ds(NS * ZCH, TAIL)], out_hbm.at[c, pl.ds(NS * ZCH, TAIL)])


@functools.partial(
    pl.kernel,
    out_type=jax.ShapeDtypeStruct((NC, N, D), jnp.float32),
    mesh=_MESH,
    scratch_types=[
        pltpu.VMEM((4, 2, 2, K), jnp.int32),
        pltpu.VMEM((NBUF, K, D), jnp.float32),
        pltpu.VMEM_SHARED((N, D), jnp.float32),
        pltpu.SemaphoreType.DMA,
        pltpu.SemaphoreType.DMA,
        pltpu.SemaphoreType.DMA,
        pltpu.SemaphoreType.DMA,
        pltpu.SemaphoreType.DMA,
        pltpu.SemaphoreType.DMA,
        pltpu.SemaphoreType.DMA,
        pltpu.SemaphoreType.DMA,
        pltpu.SemaphoreType.DMA,
        pltpu.SemaphoreType.DMA,
        pltpu.SemaphoreType.DMA,
        pltpu.SemaphoreType.DMA,
    ],
)
def _sc_aggregate(
    u_hbm, idx3_hbm, out_hbm,
    iring, rows_v, acc_sh,
    i0, i1, i2, i3, g0, g1, g2, g3, s0, s1, s2, s3,
):
    c = lax.axis_index("c")
    s = lax.axis_index("s")
    w = c * NS + s
    isem = (i0, i1, i2, i3)
    gsem = (g0, g1, g2, g3)
    ssem = (s0, s1, s2, s3)

    zero_row = jnp.zeros((16,), jnp.float32)

    def icopy(j, jslot):
        return pltpu.make_async_copy(
            idx3_hbm.at[w, pl.ds(j * 2, 2)], iring.at[jslot], isem[jslot]
        )

    def gather(rb, ps, par):
        return pltpu.make_async_copy(
            u_hbm.at[iring.at[ps, par, 0]], rows_v.at[rb], gsem[rb]
        )

    def scat(rb, ps, par):
        return pltpu.make_async_copy(
            rows_v.at[rb], acc_sh.at[iring.at[ps, par, 1]], ssem[rb]
        )

    icopy(0, 0).start()
    icopy(1, 1).start()
    icopy(2, 2).start()
    icopy(0, 0).wait()
    gather(0, 0, 0).start()
    gather(1, 0, 1).start()
    icopy(1, 1).wait()
    gather(2, 1, 0).start()

    def zfill(t, carry):
        rows_v[3, t // 8, pl.ds((t % 8) * 16, 16)] = zero_row
        return carry

    lax.fori_loop(0, ZB * 8, zfill, 0)
    _zero_acc(rows_v.at[3], acc_sh, s)
    plsc.subcore_barrier()

    def octet(i, carry):
        t0 = i * OCT
        for b8 in range(OCT):
            t = t0 + b8
            rb = b8 % NBUF
            ps = (b8 // 2) % 4
            par = b8 % 2
            gather(rb, ps, par).wait()
            scat(rb, ps, par).start(add=True)

            @pl.when(t >= 1)
            def _():
                scat((b8 - 1) % NBUF, ((b8 - 1) // 2) % 4, (b8 - 1) % 2).wait()

            if par == 0:
                j = t // 2

                @pl.when(j + 2 < NPAIRS)
                def _():
                    icopy(j + 2, (b8 // 2 + 2) % 4).wait()

                @pl.when(j + 3 < NPAIRS)
                def _():
                    icopy(j + 3, (b8 // 2 + 3) % 4).start()

            @pl.when(t + PF < STEPS)
            def _():
                gather((b8 + PF) % NBUF, ((b8 + PF) // 2) % 4, (b8 + PF) % 2).start()

        return carry

    lax.fori_loop(0, STEPS // OCT, octet, 0)
    scat((STEPS - 1) % NBUF, ((STEPS - 1) // 2) % 4, (STEPS - 1) % 2).wait()
    plsc.subcore_barrier()
    _copy_out(acc_sh, out_hbm, c, s)


@functools.partial(
    pl.kernel,
    out_type=jax.ShapeDtypeStruct((NC, N, D), jnp.float32),
    mesh=_MESH,
    scratch_types=[
        pltpu.VMEM((STEPSD, KD), jnp.int32),
        pltpu.VMEM((KD, D), jnp.float32),
        pltpu.VMEM_SHARED((N, D), jnp.float32),
        pltpu.SemaphoreType.DMA,
    ],
)
def _sc_degree(dst3_hbm, out_hbm, dst_all, ones_v, acc_sh, s0):
    c = lax.axis_index("c")
    s = lax.axis_index("s")
    w = c * NS + s
    zero_row = jnp.zeros((16,), jnp.float32)
    e1_row = jnp.where(lax.iota(jnp.int32, 16) == 0, 1.0, 0.0).astype(jnp.float32)

    _fill_rows(ones_v, KD, D, zero_row)
    _zero_acc(ones_v, acc_sh, s)

    def set_e1(i, carry):
        ones_v[i, pl.ds(0, 16)] = e1_row
        return carry

    lax.fori_loop(0, KD, set_e1, 0)
    pltpu.sync_copy(dst3_hbm.at[w], dst_all)
    plsc.subcore_barrier()

    def scat(t):
        return pltpu.make_async_copy(ones_v, acc_sh.at[dst_all.at[t]], s0)

    def batch(i, carry):
        t0 = i * FIRE

        def fire(t, cc):
            scat(t).start(add=True)
            return cc

        def drain(t, cc):
            scat(t).wait()
            return cc

        lax.fori_loop(t0, t0 + FIRE, fire, 0)
        lax.fori_loop(t0, t0 + FIRE, drain, 0)
        return carry

    lax.fori_loop(0, STEPSD // FIRE, batch, 0)
    plsc.subcore_barrier()
    _copy_out(acc_sh, out_hbm, c, s)


_BN = 2000


def _prep_body(degp_ref, x_ref, dinv_ref, u_ref):
    d = degp_ref[0, :, 0:1] + degp_ref[1, :, 0:1] + 1.0
    dv = lax.rsqrt(d)
    dinv_ref[...] = dv
    u_ref[...] = dv * x_ref[...]


def _tc_prep(degp, x):
    return pl.pallas_call(
        _prep_body,
        grid=(N // _BN,),
        in_specs=[
            pl.BlockSpec((NC, _BN, D), lambda i: (0, i, 0)),
            pl.BlockSpec((_BN, D), lambda i: (i, 0)),
        ],
        out_specs=[
            pl.BlockSpec((_BN, 1), lambda i: (i, 0)),
            pl.BlockSpec((_BN, D), lambda i: (i, 0)),
        ],
        out_shape=[
            jax.ShapeDtypeStruct((N, 1), jnp.float32),
            jax.ShapeDtypeStruct((N, D), jnp.float32),
        ],
    )(degp, x)


def _layer_body(p_ref, u_ref, dinv_ref, w_ref, b_ref, o_ref, *, scale_out):
    dv = dinv_ref[...]
    agg = (p_ref[0] + p_ref[1] + u_ref[...]) * dv
    h = jnp.dot(agg, w_ref[...], preferred_element_type=jnp.float32) + b_ref[...]
    if scale_out:
        h = jnp.maximum(h, 0.0) * dv
    o_ref[...] = h


def _tc_layer(p, u, dinv, w, b, scale_out):
    return pl.pallas_call(
        functools.partial(_layer_body, scale_out=scale_out),
        grid=(N // _BN,),
        in_specs=[
            pl.BlockSpec((NC, _BN, D), lambda i: (0, i, 0)),
            pl.BlockSpec((_BN, D), lambda i: (i, 0)),
            pl.BlockSpec((_BN, 1), lambda i: (i, 0)),
            pl.BlockSpec((D, D), lambda i: (0, 0)),
            pl.BlockSpec((1, D), lambda i: (0, 0)),
        ],
        out_specs=pl.BlockSpec((_BN, D), lambda i: (i, 0)),
        out_shape=jax.ShapeDtypeStruct((N, D), jnp.float32),
    )(p, u, dinv, w, b)


def kernel(x, edge_index, W1, b1, W2, b2, W3, b3):
    src3 = jnp.reshape(edge_index[0], (NW, STEPS, K))
    dst3 = jnp.reshape(edge_index[1], (NW, STEPS, K))
    idx3 = jnp.stack([src3, dst3], axis=2)
    dstd = jnp.reshape(edge_index[1], (NW, STEPSD, KD))
    degp = _sc_degree(dstd)
    dinv, u1 = _tc_prep(degp, x)
    p1 = _sc_aggregate(u1, idx3)
    u2 = _tc_layer(p1, u1, dinv, W1, jnp.reshape(b1, (1, D)), True)
    p2 = _sc_aggregate(u2, idx3)
    u3 = _tc_layer(p2, u2, dinv, W2, jnp.reshape(b2, (1, D)), True)
    p3 = _sc_aggregate(u3, idx3)
    w3p = jnp.zeros((D, D), jnp.float32).at[:, :NCLS].set(W3)
    b3p = jnp.zeros((1, D), jnp.float32).at[0, :NCLS].set(b3)
    out = _tc_layer(p3, u3, dinv, w3p, b3p, False)
    return out[:, :NCLS]


# speedup vs baseline: 27.3523x; 1.0031x over previous
import functools

import jax
import jax.numpy as jnp
from jax import lax
from jax.experimental import pallas as pl
from jax.experimental.pallas import tpu as pltpu
from jax.experimental.pallas import tpu_sc as plsc

N = 10000
E = 320000
D = 128
NCLS = 40

NC = 2
NS = 16
NW = NC * NS
EPW = E // NW
K = 50
STEPS = EPW // K
NPAIRS = STEPS // 2
NBUF = 5
PF = 4
OCT = 40
ZCH = 624
TAIL = N - NS * ZCH
ZB = 48
KD = 100
STEPSD = EPW // KD
FIRE = 25

_MESH = plsc.VectorSubcoreMesh(
    core_axis_name="c", subcore_axis_name="s", num_cores=NC, num_subcores=NS
)


def _fill_rows(ref, rows, width, value_row):
    per_row = width // 16

    def body(t, carry):
        i = t // per_row
        j = t % per_row
        ref[i, pl.ds(j * 16, 16)] = value_row
        return carry

    lax.fori_loop(0, rows * per_row, body, 0)


def _zero_acc(zsrc, acc_sh, s):
    for z in range(ZCH // ZB):
        pltpu.sync_copy(zsrc.at[pl.ds(0, ZB)], acc_sh.at[pl.ds(s * ZCH + z * ZB, ZB)])

    @pl.when(s == 0)
    def _():
        pltpu.sync_copy(zsrc.at[pl.ds(0, TAIL)], acc_sh.at[pl.ds(NS * ZCH, TAIL)])


def _copy_out(acc_sh, out_hbm, c, s):
    pltpu.sync_copy(acc_sh.at[pl.ds(s * ZCH, ZCH)], out_hbm.at[c, pl.ds(s * ZCH, ZCH)])

    @pl.when(s == 0)
    def _():
        pltpu.sync_copy(acc_sh.at[pl.ds(NS * ZCH, TAIL)], out_hbm.at[c, pl.ds(NS * ZCH, TAIL)])


@functools.partial(
    pl.kernel,
    out_type=jax.ShapeDtypeStruct((NC, N, D), jnp.float32),
    mesh=_MESH,
    scratch_types=[
        pltpu.VMEM((4, 2, 2, K), jnp.int32),
        pltpu.VMEM((NBUF, K, D), jnp.float32),
        pltpu.VMEM_SHARED((N, D), jnp.float32),
        pltpu.SemaphoreType.DMA,
        pltpu.SemaphoreType.DMA,
        pltpu.SemaphoreType.DMA,
        pltpu.SemaphoreType.DMA,
        pltpu.SemaphoreType.DMA,
        pltpu.SemaphoreType.DMA,
        pltpu.SemaphoreType.DMA,
        pltpu.SemaphoreType.DMA,
        pltpu.SemaphoreType.DMA,
        pltpu.SemaphoreType.DMA,
        pltpu.SemaphoreType.DMA,
        pltpu.SemaphoreType.DMA,
        pltpu.SemaphoreType.DMA,
        pltpu.SemaphoreType.DMA,
    ],
)
def _sc_aggregate(
    u_hbm, idx3_hbm, out_hbm,
    iring, rows_v, acc_sh,
    i0, i1, i2, i3, g0, g1, g2, g3, g4, s0, s1, s2, s3, s4,
):
    c = lax.axis_index("c")
    s = lax.axis_index("s")
    w = c * NS + s
    isem = (i0, i1, i2, i3)
    gsem = (g0, g1, g2, g3, g4)
    ssem = (s0, s1, s2, s3, s4)

    zero_row = jnp.zeros((16,), jnp.float32)

    def icopy(j, jslot):
        return pltpu.make_async_copy(
            idx3_hbm.at[w, pl.ds(j * 2, 2)], iring.at[jslot], isem[jslot]
        )

    def gather(rb, ps, par):
        return pltpu.make_async_copy(
            u_hbm.at[iring.at[ps, par, 0]], rows_v.at[rb], gsem[rb]
        )

    def scat(rb, ps, par):
        return pltpu.make_async_copy(
            rows_v.at[rb], acc_sh.at[iring.at[ps, par, 1]], ssem[rb]
        )

    icopy(0, 0).start()
    icopy(1, 1).start()
    icopy(2, 2).start()
    icopy(0, 0).wait()
    gather(0, 0, 0).start()
    gather(1, 0, 1).start()
    icopy(1, 1).wait()
    gather(2, 1, 0).start()

    def zfill(t, carry):
        rows_v[3, t // 8, pl.ds((t % 8) * 16, 16)] = zero_row
        return carry

    lax.fori_loop(0, ZB * 8, zfill, 0)
    _zero_acc(rows_v.at[3], acc_sh, s)
    gather(3, 1, 1).start()
    plsc.subcore_barrier()

    def octet(i, carry):
        t0 = i * OCT
        for b8 in range(OCT):
            t = t0 + b8
            rb = b8 % NBUF
            ps = (b8 // 2) % 4
            par = b8 % 2
            gather(rb, ps, par).wait()
            scat(rb, ps, par).start(add=True)

            @pl.when(t >= 1)
            def _():
                scat((b8 - 1) % NBUF, ((b8 - 1) // 2) % 4, (b8 - 1) % 2).wait()

            if par == 0:
                j = t // 2

                @pl.when(j + 2 < NPAIRS)
                def _():
                    icopy(j + 2, (b8 // 2 + 2) % 4).wait()

                @pl.when(j + 3 < NPAIRS)
                def _():
                    icopy(j + 3, (b8 // 2 + 3) % 4).start()

            @pl.when(t + PF < STEPS)
            def _():
                gather((b8 + PF) % NBUF, ((b8 + PF) // 2) % 4, (b8 + PF) % 2).start()

        return carry

    lax.fori_loop(0, STEPS // OCT, octet, 0)
    scat((STEPS - 1) % NBUF, ((STEPS - 1) // 2) % 4, (STEPS - 1) % 2).wait()
    plsc.subcore_barrier()
    _copy_out(acc_sh, out_hbm, c, s)


@functools.partial(
    pl.kernel,
    out_type=jax.ShapeDtypeStruct((NC, N, D), jnp.float32),
    mesh=_MESH,
    scratch_types=[
        pltpu.VMEM((STEPSD, KD), jnp.int32),
        pltpu.VMEM((KD, D), jnp.float32),
        pltpu.VMEM_SHARED((N, D), jnp.float32),
        pltpu.SemaphoreType.DMA,
    ],
)
def _sc_degree(dst3_hbm, out_hbm, dst_all, ones_v, acc_sh, s0):
    c = lax.axis_index("c")
    s = lax.axis_index("s")
    w = c * NS + s
    zero_row = jnp.zeros((16,), jnp.float32)
    e1_row = jnp.where(lax.iota(jnp.int32, 16) == 0, 1.0, 0.0).astype(jnp.float32)

    _fill_rows(ones_v, KD, D, zero_row)
    _zero_acc(ones_v, acc_sh, s)

    def set_e1(i, carry):
        ones_v[i, pl.ds(0, 16)] = e1_row
        return carry

    lax.fori_loop(0, KD, set_e1, 0)
    pltpu.sync_copy(dst3_hbm.at[w], dst_all)
    plsc.subcore_barrier()

    def scat(t):
        return pltpu.make_async_copy(ones_v, acc_sh.at[dst_all.at[t]], s0)

    def batch(i, carry):
        t0 = i * FIRE

        def fire(t, cc):
            scat(t).start(add=True)
            return cc

        def drain(t, cc):
            scat(t).wait()
            return cc

        lax.fori_loop(t0, t0 + FIRE, fire, 0)
        lax.fori_loop(t0, t0 + FIRE, drain, 0)
        return carry

    lax.fori_loop(0, STEPSD // FIRE, batch, 0)
    plsc.subcore_barrier()
    _copy_out(acc_sh, out_hbm, c, s)


_BN = 2000


def _prep_body(degp_ref, x_ref, dinv_ref, u_ref):
    d = degp_ref[0, :, 0:1] + degp_ref[1, :, 0:1] + 1.0
    dv = lax.rsqrt(d)
    dinv_ref[...] = dv
    u_ref[...] = dv * x_ref[...]


def _tc_prep(degp, x):
    return pl.pallas_call(
        _prep_body,
        grid=(N // _BN,),
        in_specs=[
            pl.BlockSpec((NC, _BN, D), lambda i: (0, i, 0)),
            pl.BlockSpec((_BN, D), lambda i: (i, 0)),
        ],
        out_specs=[
            pl.BlockSpec((_BN, 1), lambda i: (i, 0)),
            pl.BlockSpec((_BN, D), lambda i: (i, 0)),
        ],
        out_shape=[
            jax.ShapeDtypeStruct((N, 1), jnp.float32),
            jax.ShapeDtypeStruct((N, D), jnp.float32),
        ],
    )(degp, x)


def _layer_body(p_ref, u_ref, dinv_ref, w_ref, b_ref, o_ref, *, scale_out):
    dv = dinv_ref[...]
    agg = (p_ref[0] + p_ref[1] + u_ref[...]) * dv
    h = jnp.dot(agg, w_ref[...], preferred_element_type=jnp.float32) + b_ref[...]
    if scale_out:
        h = jnp.maximum(h, 0.0) * dv
    o_ref[...] = h


def _tc_layer(p, u, dinv, w, b, scale_out):
    return pl.pallas_call(
        functools.partial(_layer_body, scale_out=scale_out),
        grid=(N // _BN,),
        in_specs=[
            pl.BlockSpec((NC, _BN, D), lambda i: (0, i, 0)),
            pl.BlockSpec((_BN, D), lambda i: (i, 0)),
            pl.BlockSpec((_BN, 1), lambda i: (i, 0)),
            pl.BlockSpec((D, D), lambda i: (0, 0)),
            pl.BlockSpec((1, D), lambda i: (0, 0)),
        ],
        out_specs=pl.BlockSpec((_BN, D), lambda i: (i, 0)),
        out_shape=jax.ShapeDtypeStruct((N, D), jnp.float32),
    )(p, u, dinv, w, b)


def kernel(x, edge_index, W1, b1, W2, b2, W3, b3):
    src3 = jnp.reshape(edge_index[0], (NW, STEPS, K))
    dst3 = jnp.reshape(edge_index[1], (NW, STEPS, K))
    idx3 = jnp.stack([src3, dst3], axis=2)
    dstd = jnp.reshape(edge_index[1], (NW, STEPSD, KD))
    degp = _sc_degree(dstd)
    dinv, u1 = _tc_prep(degp, x)
    p1 = _sc_aggregate(u1, idx3)
    u2 = _tc_layer(p1, u1, dinv, W1, jnp.reshape(b1, (1, D)), True)
    p2 = _sc_aggregate(u2, idx3)
    u3 = _tc_layer(p2, u2, dinv, W2, jnp.reshape(b2, (1, D)), True)
    p3 = _sc_aggregate(u3, idx3)
    w3p = jnp.zeros((D, D), jnp.float32).at[:, :NCLS].set(W3)
    b3p = jnp.zeros((1, D), jnp.float32).at[0, :NCLS].set(b3)
    out = _tc_layer(p3, u3, dinv, w3p, b3p, False)
    return out[:, :NCLS]


# speedup vs baseline: 27.4579x; 1.0039x over previous
import functools

import jax
import jax.numpy as jnp
from jax import lax
from jax.experimental import pallas as pl
from jax.experimental.pallas import tpu as pltpu
from jax.experimental.pallas import tpu_sc as plsc

N = 10000
E = 320000
D = 128
NCLS = 40

NC = 2
NS = 16
NW = NC * NS
EPW = E // NW
K = 50
STEPS = EPW // K
NPAIRS = STEPS // 2
NBUF = 5
PF = 4
OCT = 40
ZCH = 624
TAIL = N - NS * ZCH
ZB = 48
KD = 100
STEPSD = EPW // KD
DW = 128
FIRE = 25

_MESH = plsc.VectorSubcoreMesh(
    core_axis_name="c", subcore_axis_name="s", num_cores=NC, num_subcores=NS
)


def _fill_rows(ref, rows, width, value_row):
    per_row = width // 16

    def body(t, carry):
        i = t // per_row
        j = t % per_row
        ref[i, pl.ds(j * 16, 16)] = value_row
        return carry

    lax.fori_loop(0, rows * per_row, body, 0)


def _zero_acc(zsrc, acc_sh, s):
    for z in range(ZCH // ZB):
        pltpu.sync_copy(zsrc.at[pl.ds(0, ZB)], acc_sh.at[pl.ds(s * ZCH + z * ZB, ZB)])

    @pl.when(s == 0)
    def _():
        pltpu.sync_copy(zsrc.at[pl.ds(0, TAIL)], acc_sh.at[pl.ds(NS * ZCH, TAIL)])


def _copy_out(acc_sh, out_hbm, c, s):
    pltpu.sync_copy(acc_sh.at[pl.ds(s * ZCH, ZCH)], out_hbm.at[c, pl.ds(s * ZCH, ZCH)])

    @pl.when(s == 0)
    def _():
        pltpu.sync_copy(acc_sh.at[pl.ds(NS * ZCH, TAIL)], out_hbm.at[c, pl.ds(NS * ZCH, TAIL)])


@functools.partial(
    pl.kernel,
    out_type=jax.ShapeDtypeStruct((NC, N, D), jnp.float32),
    mesh=_MESH,
    scratch_types=[
        pltpu.VMEM((4, 2, 2, K), jnp.int32),
        pltpu.VMEM((NBUF, K, D), jnp.float32),
        pltpu.VMEM_SHARED((N, D), jnp.float32),
        pltpu.SemaphoreType.DMA,
        pltpu.SemaphoreType.DMA,
        pltpu.SemaphoreType.DMA,
        pltpu.SemaphoreType.DMA,
        pltpu.SemaphoreType.DMA,
        pltpu.SemaphoreType.DMA,
        pltpu.SemaphoreType.DMA,
        pltpu.SemaphoreType.DMA,
        pltpu.SemaphoreType.DMA,
        pltpu.SemaphoreType.DMA,
        pltpu.SemaphoreType.DMA,
        pltpu.SemaphoreType.DMA,
        pltpu.SemaphoreType.DMA,
        pltpu.SemaphoreType.DMA,
    ],
)
def _sc_aggregate(
    u_hbm, idx3_hbm, out_hbm,
    iring, rows_v, acc_sh,
    i0, i1, i2, i3, g0, g1, g2, g3, g4, s0, s1, s2, s3, s4,
):
    c = lax.axis_index("c")
    s = lax.axis_index("s")
    w = c * NS + s
    isem = (i0, i1, i2, i3)
    gsem = (g0, g1, g2, g3, g4)
    ssem = (s0, s1, s2, s3, s4)

    zero_row = jnp.zeros((16,), jnp.float32)

    def icopy(j, jslot):
        return pltpu.make_async_copy(
            idx3_hbm.at[w, pl.ds(j * 2, 2)], iring.at[jslot], isem[jslot]
        )

    def gather(rb, ps, par):
        return pltpu.make_async_copy(
            u_hbm.at[iring.at[ps, par, 0]], rows_v.at[rb], gsem[rb]
        )

    def scat(rb, ps, par):
        return pltpu.make_async_copy(
            rows_v.at[rb], acc_sh.at[iring.at[ps, par, 1]], ssem[rb]
        )

    icopy(0, 0).start()
    icopy(1, 1).start()
    icopy(2, 2).start()
    icopy(0, 0).wait()
    gather(0, 0, 0).start()
    gather(1, 0, 1).start()
    icopy(1, 1).wait()
    gather(2, 1, 0).start()

    def zfill(t, carry):
        rows_v[3, t // 8, pl.ds((t % 8) * 16, 16)] = zero_row
        return carry

    lax.fori_loop(0, ZB * 8, zfill, 0)
    _zero_acc(rows_v.at[3], acc_sh, s)
    gather(3, 1, 1).start()
    plsc.subcore_barrier()

    def octet(i, carry):
        t0 = i * OCT
        for b8 in range(OCT):
            t = t0 + b8
            rb = b8 % NBUF
            ps = (b8 // 2) % 4
            par = b8 % 2
            gather(rb, ps, par).wait()
            scat(rb, ps, par).start(add=True)

            @pl.when(t >= 1)
            def _():
                scat((b8 - 1) % NBUF, ((b8 - 1) // 2) % 4, (b8 - 1) % 2).wait()

            if par == 0:
                j = t // 2

                @pl.when(j + 2 < NPAIRS)
                def _():
                    icopy(j + 2, (b8 // 2 + 2) % 4).wait()

                @pl.when(j + 3 < NPAIRS)
                def _():
                    icopy(j + 3, (b8 // 2 + 3) % 4).start()

            @pl.when(t + PF < STEPS)
            def _():
                gather((b8 + PF) % NBUF, ((b8 + PF) // 2) % 4, (b8 + PF) % 2).start()

        return carry

    lax.fori_loop(0, STEPS // OCT, octet, 0)
    scat((STEPS - 1) % NBUF, ((STEPS - 1) // 2) % 4, (STEPS - 1) % 2).wait()
    plsc.subcore_barrier()
    _copy_out(acc_sh, out_hbm, c, s)


@functools.partial(
    pl.kernel,
    out_type=jax.ShapeDtypeStruct((NC, N, DW), jnp.float32),
    mesh=_MESH,
    scratch_types=[
        pltpu.VMEM((STEPSD, KD), jnp.int32),
        pltpu.VMEM((KD, DW), jnp.float32),
        pltpu.VMEM_SHARED((N, DW), jnp.float32),
        pltpu.SemaphoreType.DMA,
    ],
)
def _sc_degree(dst3_hbm, out_hbm, dst_all, ones_v, acc_sh, s0):
    c = lax.axis_index("c")
    s = lax.axis_index("s")
    w = c * NS + s
    zero_row = jnp.zeros((16,), jnp.float32)
    e1_row = jnp.where(lax.iota(jnp.int32, 16) == 0, 1.0, 0.0).astype(jnp.float32)

    _fill_rows(ones_v, KD, DW, zero_row)
    _zero_acc(ones_v, acc_sh, s)

    def set_e1(i, carry):
        ones_v[i, pl.ds(0, 16)] = e1_row
        return carry

    lax.fori_loop(0, KD, set_e1, 0)
    pltpu.sync_copy(dst3_hbm.at[w], dst_all)
    plsc.subcore_barrier()

    def scat(t):
        return pltpu.make_async_copy(ones_v, acc_sh.at[dst_all.at[t]], s0)

    def batch(i, carry):
        t0 = i * FIRE

        def fire(t, cc):
            scat(t).start(add=True)
            return cc

        def drain(t, cc):
            scat(t).wait()
            return cc

        lax.fori_loop(t0, t0 + FIRE, fire, 0)
        lax.fori_loop(t0, t0 + FIRE, drain, 0)
        return carry

    lax.fori_loop(0, STEPSD // FIRE, batch, 0)
    plsc.subcore_barrier()
    _copy_out(acc_sh, out_hbm, c, s)


_BN = 2000


def _prep_body(degp_ref, x_ref, dinv_ref, u_ref):
    d = degp_ref[0, :, 0:1] + degp_ref[1, :, 0:1] + 1.0
    dv = lax.rsqrt(d)
    dinv_ref[...] = dv
    u_ref[...] = dv * x_ref[...]


def _tc_prep(degp, x):
    return pl.pallas_call(
        _prep_body,
        grid=(N // _BN,),
        in_specs=[
            pl.BlockSpec((NC, _BN, DW), lambda i: (0, i, 0)),
            pl.BlockSpec((_BN, D), lambda i: (i, 0)),
        ],
        out_specs=[
            pl.BlockSpec((_BN, 1), lambda i: (i, 0)),
            pl.BlockSpec((_BN, D), lambda i: (i, 0)),
        ],
        out_shape=[
            jax.ShapeDtypeStruct((N, 1), jnp.float32),
            jax.ShapeDtypeStruct((N, D), jnp.float32),
        ],
    )(degp, x)


def _mm_body(x_ref, w_ref, o_ref):
    o_ref[...] = jnp.dot(x_ref[...], w_ref[...], preferred_element_type=jnp.float32)


def _tc_mm(x, w):
    return pl.pallas_call(
        _mm_body,
        grid=(N // _BN,),
        in_specs=[
            pl.BlockSpec((_BN, D), lambda i: (i, 0)),
            pl.BlockSpec((D, D), lambda i: (0, 0)),
        ],
        out_specs=pl.BlockSpec((_BN, D), lambda i: (i, 0)),
        out_shape=jax.ShapeDtypeStruct((N, D), jnp.float32),
    )(x, w)


def _layer_body(p_ref, u_ref, dinv_ref, w_ref, b_ref, o_ref):
    dv = dinv_ref[...]
    h = jnp.maximum((p_ref[0] + p_ref[1] + u_ref[...]) * dv + b_ref[...], 0.0)
    o_ref[...] = jnp.dot(h, w_ref[...], preferred_element_type=jnp.float32) * dv


def _final_body(p_ref, u_ref, dinv_ref, b_ref, o_ref):
    o_ref[...] = (p_ref[0] + p_ref[1] + u_ref[...]) * dinv_ref[...] + b_ref[...]


def _tc_layer(p, u, dinv, w, b):
    return pl.pallas_call(
        _layer_body,
        grid=(N // _BN,),
        in_specs=[
            pl.BlockSpec((NC, _BN, D), lambda i: (0, i, 0)),
            pl.BlockSpec((_BN, D), lambda i: (i, 0)),
            pl.BlockSpec((_BN, 1), lambda i: (i, 0)),
            pl.BlockSpec((D, D), lambda i: (0, 0)),
            pl.BlockSpec((1, D), lambda i: (0, 0)),
        ],
        out_specs=pl.BlockSpec((_BN, D), lambda i: (i, 0)),
        out_shape=jax.ShapeDtypeStruct((N, D), jnp.float32),
    )(p, u, dinv, w, b)


def _tc_final(p, u, dinv, b):
    return pl.pallas_call(
        _final_body,
        grid=(N // _BN,),
        in_specs=[
            pl.BlockSpec((NC, _BN, D), lambda i: (0, i, 0)),
            pl.BlockSpec((_BN, D), lambda i: (i, 0)),
            pl.BlockSpec((_BN, 1), lambda i: (i, 0)),
            pl.BlockSpec((1, D), lambda i: (0, 0)),
        ],
        out_specs=pl.BlockSpec((_BN, D), lambda i: (i, 0)),
        out_shape=jax.ShapeDtypeStruct((N, D), jnp.float32),
    )(p, u, dinv, b)


def kernel(x, edge_index, W1, b1, W2, b2, W3, b3):
    src3 = jnp.reshape(edge_index[0], (NW, STEPS, K))
    dst3 = jnp.reshape(edge_index[1], (NW, STEPS, K))
    idx3 = jnp.stack([src3, dst3], axis=2)
    dstd = jnp.reshape(edge_index[1], (NW, STEPSD, KD))
    z1 = _tc_mm(x, W1)
    degp = _sc_degree(dstd)
    dinv, u1 = _tc_prep(degp, z1)
    p1 = _sc_aggregate(u1, idx3)
    u2 = _tc_layer(p1, u1, dinv, W2, jnp.reshape(b1, (1, D)))
    p2 = _sc_aggregate(u2, idx3)
    w3p = jnp.zeros((D, D), jnp.float32).at[:, :NCLS].set(W3)
    u3 = _tc_layer(p2, u2, dinv, w3p, jnp.reshape(b2, (1, D)))
    p3 = _sc_aggregate(u3, idx3)
    b3p = jnp.zeros((1, D), jnp.float32).at[0, :NCLS].set(b3)
    out = _tc_final(p3, u3, dinv, b3p)
    return out[:, :NCLS]
